# baseline (speedup 1.0000x reference)
import functools

import jax
import jax.numpy as jnp
from jax import lax
from jax.experimental import pallas as pl
from jax.experimental.pallas import tpu as pltpu
from jax.experimental.pallas import tpu_sc as plsc

_D = 64
_NC = 2
_NS = 16
_NW = _NC * _NS
_CHUNK = 256
_NBUF = 4


def _build_gather(B: int):
    b_per_w = B // _NW
    nchunk = b_per_w // _CHUNK
    ngroup = nchunk // _NBUF
    assert b_per_w * _NW == B and nchunk * _CHUNK == b_per_w
    assert ngroup * _NBUF == nchunk and ngroup >= 2

    mesh = plsc.VectorSubcoreMesh(core_axis_name="c", subcore_axis_name="s")

    @functools.partial(
        pl.kernel,
        out_type=jax.ShapeDtypeStruct((B, _D), jnp.float32),
        mesh=mesh,
        compiler_params=pltpu.CompilerParams(use_tc_tiling_on_sc=False),
        scratch_types=[
            pltpu.VMEM((nchunk, _CHUNK), jnp.int32),
            [pltpu.VMEM((_CHUNK, _D), jnp.float32) for _ in range(_NBUF)],
            [pltpu.SemaphoreType.DMA for _ in range(_NBUF)],
            [pltpu.SemaphoreType.DMA for _ in range(_NBUF)],
        ],
    )
    def gather_kernel(idx_hbm, table_hbm, out_hbm, idx_v, rows, gsem, ssem):
        wid = lax.axis_index("s") * _NC + lax.axis_index("c")
        base = wid * b_per_w
        pltpu.sync_copy(idx_hbm.at[wid], idx_v)

        def start_gather(g, b):
            pltpu.async_copy(table_hbm.at[idx_v.at[g]], rows[b], gsem[b])

        def wait_gather(g, b):
            pltpu.make_async_copy(
                table_hbm.at[idx_v.at[g]], rows[b], gsem[b]).wait()

        def out_slice(g):
            return out_hbm.at[pl.ds(base + g * _CHUNK, _CHUNK)]

        def start_store(g, b):
            pltpu.async_copy(rows[b], out_slice(g), ssem[b])

        def wait_store(g, b):
            pltpu.make_async_copy(rows[b], out_slice(g), ssem[b]).wait()

        for b in range(_NBUF):
            start_gather(b, b)

        def body(i, carry):
            g0 = i * _NBUF
            for b in range(_NBUF):
                wait_gather(g0 + b, b)
                start_store(g0 + b, b)
            for b in range(_NBUF):
                wait_store(g0 + b, b)
                start_gather(g0 + _NBUF + b, b)
            return carry

        lax.fori_loop(0, ngroup - 1, body, 0)

        g0 = (ngroup - 1) * _NBUF
        for b in range(_NBUF):
            wait_gather(g0 + b, b)
            start_store(g0 + b, b)
        for b in range(_NBUF):
            wait_store(g0 + b, b)

    return gather_kernel


_B_FIXED = 16384 * 26
_GATHER = _build_gather(_B_FIXED)


def kernel(token_ids, weight):
    s, t = token_ids.shape
    b = s * t
    idx = token_ids.reshape(_NW, b // _NW // _CHUNK, _CHUNK).astype(jnp.int32)
    out = _GATHER(idx, weight)
    return out.reshape(s, t, _D)


# speedup vs baseline: 1.0045x; 1.0045x over previous
import functools

import jax
import jax.numpy as jnp
from jax import lax
from jax.experimental import pallas as pl
from jax.experimental.pallas import tpu as pltpu
from jax.experimental.pallas import tpu_sc as plsc

_D = 64
_NC = 2
_NS = 16
_NW = _NC * _NS
_CHUNK = 256
_NBUF = 4
_LEAD = 2


def _build_gather(B: int):
    b_per_w = B // _NW
    nchunk = b_per_w // _CHUNK
    ngroup = nchunk // _NBUF
    assert b_per_w * _NW == B and nchunk * _CHUNK == b_per_w
    assert ngroup * _NBUF == nchunk and ngroup >= 2

    mesh = plsc.VectorSubcoreMesh(core_axis_name="c", subcore_axis_name="s")

    @functools.partial(
        pl.kernel,
        out_type=jax.ShapeDtypeStruct((B, _D), jnp.float32),
        mesh=mesh,
        compiler_params=pltpu.CompilerParams(use_tc_tiling_on_sc=False),
        scratch_types=[
            pltpu.VMEM((nchunk, _CHUNK), jnp.int32),
            [pltpu.VMEM((_CHUNK, _D), jnp.float32) for _ in range(_NBUF)],
            [pltpu.SemaphoreType.DMA for _ in range(_NBUF)],
            [pltpu.SemaphoreType.DMA for _ in range(_NBUF)],
        ],
    )
    def gather_kernel(idx_hbm, table_hbm, out_hbm, idx_v, rows, gsem, ssem):
        wid = lax.axis_index("s") * _NC + lax.axis_index("c")
        base = wid * b_per_w
        pltpu.sync_copy(idx_hbm.at[wid], idx_v)

        def start_gather(g, b):
            pltpu.async_copy(table_hbm.at[idx_v.at[g]], rows[b], gsem[b])

        def wait_gather(g, b):
            pltpu.make_async_copy(
                table_hbm.at[idx_v.at[g]], rows[b], gsem[b]).wait()

        def out_slice(g):
            return out_hbm.at[pl.ds(base + g * _CHUNK, _CHUNK)]

        def start_store(g, b):
            pltpu.async_copy(rows[b], out_slice(g), ssem[b])

        def wait_store(g, b):
            pltpu.make_async_copy(rows[b], out_slice(g), ssem[b]).wait()

        for b in range(_NBUF):
            start_gather(b, b)

        def body(i, carry):
            g0 = i * _NBUF
            for b in range(_NBUF):
                g = g0 + b
                wait_gather(g, b)
                start_store(g, b)
                bn = (b + _LEAD) % _NBUF

                @pl.when(jnp.logical_and(g >= _LEAD, g + _LEAD < nchunk))
                def _():
                    wait_store(g - (_NBUF - _LEAD), bn)
                    start_gather(g + _LEAD, bn)

            return carry

        lax.fori_loop(0, ngroup, body, 0)

        for g in range(nchunk - _NBUF, nchunk):
            wait_store(g, g % _NBUF)

    return gather_kernel


_B_FIXED = 16384 * 26
_GATHER = _build_gather(_B_FIXED)


def kernel(token_ids, weight):
    s, t = token_ids.shape
    b = s * t
    idx = token_ids.reshape(_NW, b // _NW // _CHUNK, _CHUNK).astype(jnp.int32)
    out = _GATHER(idx, weight)
    return out.reshape(s, t, _D)
